# baseline (speedup 1.0000x reference)
import functools

import jax
import jax.numpy as jnp
from jax import lax
from jax.experimental import pallas as pl
from jax.experimental.pallas import tpu as pltpu
from jax.experimental.pallas import tpu_sc as plsc

D_MODEL = 256
N_SITES = 512
BATCH = 1024

_NUM_CORES = 2
_NUM_SUBCORES = 16
_LANES = 16
_NW = _NUM_CORES * _NUM_SUBCORES
_ROWS_PER_W = BATCH // _NW
_CHUNK = 64
_CPR = N_SITES // _CHUNK
_NCHUNK = _ROWS_PER_W * _CPR
_NBUF = 4
_NGROUP = _NCHUNK // _NBUF
_TGROUPS = _CHUNK // _LANES


def _make_sc_embed():
    mesh = plsc.VectorSubcoreMesh(core_axis_name="c", subcore_axis_name="s")

    @functools.partial(
        pl.kernel,
        mesh=mesh,
        out_type=jax.ShapeDtypeStruct((BATCH, N_SITES, D_MODEL), jnp.float32),
        scratch_types=[
            pltpu.VMEM((_ROWS_PER_W, 2 * N_SITES), jnp.int32),
            pltpu.VMEM((4, D_MODEL), jnp.float32),
            pltpu.VMEM((_CHUNK,), jnp.int32),
        ]
        + [pltpu.VMEM((_CHUNK, D_MODEL), jnp.float32) for _ in range(_NBUF)]
        + [pltpu.SemaphoreType.DMA for _ in range(_NBUF)],
    )
    def sc_embed(n_hbm, table_hbm, out_hbm, slab_v, table_v, tok_s, *bufs):
        rows_v = bufs[:_NBUF]
        w_sem = bufs[_NBUF:]
        wid = lax.axis_index("s") * _NUM_CORES + lax.axis_index("c")

        pltpu.sync_copy(table_hbm, table_v)
        pltpu.sync_copy(n_hbm.at[pl.ds(wid * _ROWS_PER_W, _ROWS_PER_W)], slab_v)

        def expand_chunk(c, k):
            r = c // _CPR
            o = (c % _CPR) * _CHUNK
            for g in range(_TGROUPS):
                dn = slab_v[r, pl.ds(o + g * _LANES, _LANES)]
                up = slab_v[r, pl.ds(N_SITES + o + g * _LANES, _LANES)]
                tok_s[pl.ds(g * _LANES, _LANES)] = up + dn + dn

            def g_body(gi, carry):
                tv = tok_s[pl.ds(gi * _LANES, _LANES)]
                for j in range(_LANES):
                    tok = tv[j]
                    t = gi * _LANES + j
                    for i in range(D_MODEL // _LANES):
                        rows_v[k][t, pl.ds(i * _LANES, _LANES)] = (
                            table_v[tok, pl.ds(i * _LANES, _LANES)])
                return carry

            lax.fori_loop(0, _TGROUPS, g_body, 0)

        def out_view(c):
            gb = wid * _ROWS_PER_W + c // _CPR
            return out_hbm.at[gb, pl.ds((c % _CPR) * _CHUNK, _CHUNK)]

        def fire_wb(c, k):
            pltpu.async_copy(rows_v[k], out_view(c), w_sem[k])

        def wait_wb(c, k):
            pltpu.make_async_copy(rows_v[k], out_view(c), w_sem[k]).wait()

        def group_body(g, carry):
            for k in range(_NBUF):
                c = g * _NBUF + k

                def drain(c=c, k=k):
                    wait_wb(c - _NBUF, k)

                pl.when(g > 0)(drain)
                expand_chunk(c, k)
                fire_wb(c, k)
            return carry

        lax.fori_loop(0, _NGROUP, group_body, 0)
        for k in range(_NBUF):
            wait_wb(_NCHUNK - _NBUF + k, k)

    return sc_embed


_sc_embed = _make_sc_embed()


def kernel(n_flat, embed_table):
    n = jnp.asarray(n_flat, jnp.int32)
    table = jnp.asarray(embed_table, jnp.float32)
    return _sc_embed(n, table)
